# baseline (speedup 1.0000x reference)
import functools

import jax
import jax.numpy as jnp
from jax import lax
from jax.experimental import pallas as pl
from jax.experimental.pallas import tpu as pltpu
from jax.experimental.pallas import tpu_sc as plsc

RACE_CARD = 1000
ETH_CARD = 1000
D = 32
BATCH = 16384

NUM_CORES = 2
NUM_SUBCORES = 16
LANES = 16
NW = NUM_CORES * NUM_SUBCORES
B_PER_W = BATCH // NW
ICHUNK = 64
N_ICHUNK = B_PER_W // ICHUNK


def _make_kernel():
    mesh = plsc.VectorSubcoreMesh(core_axis_name="c", subcore_axis_name="s")

    @functools.partial(
        pl.kernel,
        mesh=mesh,
        compiler_params=pltpu.CompilerParams(needs_layout_passes=False),
        out_type=jax.ShapeDtypeStruct((3 * D * BATCH,), jnp.float32),
        scratch_types=[
            pltpu.VMEM((D * RACE_CARD,), jnp.float32),
            pltpu.VMEM((D * ETH_CARD,), jnp.float32),
            pltpu.VMEM((B_PER_W,), jnp.int32),
            pltpu.VMEM((B_PER_W,), jnp.int32),
            pltpu.VMEM((B_PER_W,), jnp.int32),
            pltpu.VMEM((B_PER_W,), jnp.int32),
            pltpu.VMEM((1, ICHUNK, 128), jnp.float32),
            pltpu.VMEM((3 * D * B_PER_W,), jnp.float32),
            pltpu.SemaphoreType.DMA,
            pltpu.SemaphoreType.DMA,
            pltpu.SemaphoreType.DMA,
        ],
    )
    def k(xr_hbm, xe_hbm, racef_hbm, ethf_hbm, ipack_hbm, out_hbm,
          race_v, eth_v, r_idx, e_idx, b_idx, s_v, istage, out_v,
          sem_in, sem_g, sem_out):
        wid = lax.axis_index("s") * NUM_CORES + lax.axis_index("c")
        base = wid * B_PER_W

        ct_r = pltpu.async_copy(racef_hbm, race_v, sem_in)
        ct_e = pltpu.async_copy(ethf_hbm, eth_v, sem_in)
        pltpu.sync_copy(xr_hbm.at[pl.ds(base, B_PER_W)], r_idx)
        pltpu.sync_copy(xe_hbm.at[pl.ds(base, B_PER_W)], e_idx)

        def idx_body(j, carry):
            sl = pl.ds(j * LANES, LANES)
            ii = r_idx[sl] * ETH_CARD + e_idx[sl]
            b_idx[sl] = lax.shift_right_logical(ii, 2)
            s_v[sl] = lax.bitwise_and(ii, 3)
            return carry

        lax.fori_loop(0, B_PER_W // LANES, idx_body, 0, unroll=4)

        lanes = lax.iota(jnp.int32, LANES)
        zeros = jnp.zeros((LANES,), jnp.int32)

        def ichunk_body(t, carry):
            sl = pl.ds(t * ICHUNK, ICHUNK)
            pltpu.async_copy(
                ipack_hbm.at[b_idx.at[sl]], istage.at[0], sem_g).wait()

            def blk_body(b, carry2):
                g0 = t * ICHUNK + b * LANES
                rows = lanes + b * LANES
                colb = s_v[pl.ds(g0, LANES)] * D
                for c in range(D):
                    v = plsc.load_gather(istage, [zeros, rows, colb + c])
                    out_v[pl.ds((2 * D + c) * B_PER_W + g0, LANES)] = v
                return carry2

            lax.fori_loop(0, ICHUNK // LANES, blk_body, 0)
            return carry

        lax.fori_loop(0, N_ICHUNK, ichunk_body, 0)

        ct_r.wait()
        ct_e.wait()

        def small_body(j, carry):
            sl = pl.ds(j * LANES, LANES)
            r_vec = r_idx[sl]
            e_vec = e_idx[sl]
            for f in range(D):
                out_v[pl.ds(f * B_PER_W + j * LANES, LANES)] = (
                    plsc.load_gather(race_v, [r_vec * D + f]))
                out_v[pl.ds((D + f) * B_PER_W + j * LANES, LANES)] = (
                    plsc.load_gather(eth_v, [e_vec * D + f]))
            return carry

        lax.fori_loop(0, B_PER_W // LANES, small_body, 0)

        copies = []
        for f in range(3 * D):
            copies.append(pltpu.async_copy(
                out_v.at[pl.ds(f * B_PER_W, B_PER_W)],
                out_hbm.at[pl.ds(f * BATCH + base, B_PER_W)], sem_out))
        for c in copies:
            c.wait()

    return k


_sc_kernel = _make_kernel()


@jax.jit
def kernel(x, race_table, ethnicity_table, interaction_table):
    x = x.astype(jnp.int32)
    xr = x[:, 0]
    xe = x[:, 1]
    racef = race_table.reshape(-1)
    ethf = ethnicity_table.reshape(-1)
    ipack = interaction_table.reshape(RACE_CARD * ETH_CARD // 4, 4 * D)
    out = _sc_kernel(xr, xe, racef, ethf, ipack)
    return out.reshape(3 * D, BATCH).T
